# Initial kernel scaffold; baseline (speedup 1.0000x reference)
import jax
import jax.numpy as jnp
from jax.experimental import pallas as pl


def kernel(ids, ids_table, pos_table):
    raise NotImplementedError("write your pallas kernel here")


# speedup vs baseline: 8.6355x; 8.6355x over previous
import functools

import jax
import jax.numpy as jnp
from jax import lax
from jax.experimental import pallas as pl
from jax.experimental.pallas import tpu as pltpu
from jax.experimental.pallas import tpu_sc as plsc

VOCAB_SIZE = 100000
HIDDEN = 128
BATCH = 4096
SEQ = 200
MAX_POS = 512

NC = 2
NS = 16
NW = NC * NS

TOTAL = BATCH * SEQ
ROWS_PER_W = TOTAL // NW
CHUNK = 128
CHUNKS_PER_W = ROWS_PER_W // CHUNK
NBUF = 4
GROUPS = CHUNKS_PER_W // NBUF


def _body(ids_hbm, tab_hbm, pos_hbm, out_hbm, pos_out_hbm,
          idx_v, b0, b1, b2, b3, gsem, wsem):
    bufs = (b0, b1, b2, b3)
    c = lax.axis_index("c")
    s = lax.axis_index("s")
    wid = s * NC + c

    @pl.when(wid == 0)
    def _():
        pltpu.sync_copy(pos_hbm.at[pl.ds(0, 128)], b0)
        pltpu.sync_copy(b0, pos_out_hbm.at[pl.ds(0, 128)])

    @pl.when(wid == 1)
    def _():
        pltpu.sync_copy(pos_hbm.at[pl.ds(128, 72)], b0.at[pl.ds(0, 72)])
        pltpu.sync_copy(b0.at[pl.ds(0, 72)], pos_out_hbm.at[pl.ds(128, 72)])

    pltpu.sync_copy(ids_hbm.at[pl.ds(wid * CHUNKS_PER_W, CHUNKS_PER_W)], idx_v)

    out_base = wid * ROWS_PER_W

    def group(g, carry):
        for b in range(NBUF):
            j = g * NBUF + b
            pltpu.async_copy(tab_hbm.at[idx_v.at[j]], bufs[b], gsem)
        for b in range(NBUF):
            pltpu.make_async_copy(tab_hbm.at[idx_v.at[0]], bufs[b], gsem).wait()
        for b in range(NBUF):
            j = g * NBUF + b
            pltpu.async_copy(bufs[b], out_hbm.at[pl.ds(out_base + j * CHUNK, CHUNK)], wsem)
        for b in range(NBUF):
            pltpu.make_async_copy(bufs[b], out_hbm.at[pl.ds(out_base, CHUNK)], wsem).wait()
        return carry

    lax.fori_loop(0, GROUPS, group, 0)


@functools.partial(jax.jit, static_argnums=())
def kernel(ids, ids_table, pos_table):
    ids_flat = ids.reshape(TOTAL // CHUNK, CHUNK).astype(jnp.int32)
    mesh = plsc.VectorSubcoreMesh(core_axis_name="c", subcore_axis_name="s")
    run = pl.kernel(
        _body,
        out_type=(
            jax.ShapeDtypeStruct((TOTAL, HIDDEN), jnp.float32),
            jax.ShapeDtypeStruct((SEQ, HIDDEN), jnp.float32),
        ),
        mesh=mesh,
        scratch_types=[
            pltpu.VMEM((CHUNKS_PER_W, CHUNK), jnp.int32),
            pltpu.VMEM((CHUNK, HIDDEN), jnp.float32),
            pltpu.VMEM((CHUNK, HIDDEN), jnp.float32),
            pltpu.VMEM((CHUNK, HIDDEN), jnp.float32),
            pltpu.VMEM((CHUNK, HIDDEN), jnp.float32),
            pltpu.SemaphoreType.DMA,
            pltpu.SemaphoreType.DMA,
        ],
    )
    out, pos_out = run(ids_flat, ids_table, pos_table)
    return (out.reshape(BATCH, SEQ, HIDDEN), pos_out.reshape(1, SEQ, HIDDEN))


# speedup vs baseline: 9.1067x; 1.0546x over previous
import functools

import jax
import jax.numpy as jnp
from jax import lax
from jax.experimental import pallas as pl
from jax.experimental.pallas import tpu as pltpu
from jax.experimental.pallas import tpu_sc as plsc

VOCAB_SIZE = 100000
HIDDEN = 128
BATCH = 4096
SEQ = 200
MAX_POS = 512

NC = 2
NS = 16
NW = NC * NS

TOTAL = BATCH * SEQ
ROWS_PER_W = TOTAL // NW
CHUNK = 128
CHUNKS_PER_W = ROWS_PER_W // CHUNK
K = 2
GROUP_ROWS = K * CHUNK
NGROUPS = CHUNKS_PER_W // K
NPAIRS = NGROUPS // 2


def _body(ids_hbm, tab_hbm, pos_hbm, out_hbm, pos_out_hbm,
          idx_v, buf_a, buf_b, gsem, wsem):
    c = lax.axis_index("c")
    s = lax.axis_index("s")
    wid = s * NC + c
    b0 = buf_a

    @pl.when(wid == 0)
    def _():
        pltpu.sync_copy(pos_hbm.at[pl.ds(0, 128)], b0.at[pl.ds(0, 128)])
        pltpu.sync_copy(b0.at[pl.ds(0, 128)], pos_out_hbm.at[pl.ds(0, 128)])

    @pl.when(wid == 1)
    def _():
        pltpu.sync_copy(pos_hbm.at[pl.ds(128, 72)], b0.at[pl.ds(0, 72)])
        pltpu.sync_copy(b0.at[pl.ds(0, 72)], pos_out_hbm.at[pl.ds(128, 72)])

    pltpu.sync_copy(ids_hbm.at[pl.ds(wid * CHUNKS_PER_W, CHUNKS_PER_W)], idx_v)

    out_base = wid * ROWS_PER_W

    def fire_gather(g, buf):
        for k in range(K):
            pltpu.async_copy(tab_hbm.at[idx_v.at[g * K + k]],
                             buf.at[pl.ds(k * CHUNK, CHUNK)], gsem)

    def wait_gather(buf):
        for k in range(K):
            pltpu.make_async_copy(tab_hbm.at[idx_v.at[0]],
                                  buf.at[pl.ds(0, CHUNK)], gsem).wait()

    def fire_write(g, buf):
        pltpu.async_copy(buf, out_hbm.at[pl.ds(out_base + g * GROUP_ROWS, GROUP_ROWS)],
                         wsem)

    def wait_write(buf):
        pltpu.make_async_copy(buf, out_hbm.at[pl.ds(out_base, GROUP_ROWS)], wsem).wait()

    fire_gather(0, buf_a)

    def pair(p, carry):
        g0 = 2 * p
        wait_gather(buf_a)
        fire_write(g0, buf_a)
        fire_gather(g0 + 1, buf_b)
        wait_write(buf_a)
        wait_gather(buf_b)
        fire_write(g0 + 1, buf_b)

        @pl.when(g0 + 2 < NGROUPS)
        def _():
            fire_gather(g0 + 2, buf_a)

        wait_write(buf_b)
        return carry

    lax.fori_loop(0, NPAIRS, pair, 0)


@functools.partial(jax.jit, static_argnums=())
def kernel(ids, ids_table, pos_table):
    ids_flat = ids.reshape(TOTAL // CHUNK, CHUNK).astype(jnp.int32)
    mesh = plsc.VectorSubcoreMesh(core_axis_name="c", subcore_axis_name="s")
    run = pl.kernel(
        _body,
        out_type=(
            jax.ShapeDtypeStruct((TOTAL, HIDDEN), jnp.float32),
            jax.ShapeDtypeStruct((SEQ, HIDDEN), jnp.float32),
        ),
        mesh=mesh,
        scratch_types=[
            pltpu.VMEM((CHUNKS_PER_W, CHUNK), jnp.int32),
            pltpu.VMEM((GROUP_ROWS, HIDDEN), jnp.float32),
            pltpu.VMEM((GROUP_ROWS, HIDDEN), jnp.float32),
            pltpu.SemaphoreType.DMA,
            pltpu.SemaphoreType.DMA,
        ],
    )
    out, pos_out = run(ids_flat, ids_table, pos_table)
    return (out.reshape(BATCH, SEQ, HIDDEN), pos_out.reshape(1, SEQ, HIDDEN))
